# baseline (speedup 1.0000x reference)
import functools

import jax
import jax.numpy as jnp
from jax import lax
from jax.experimental import pallas as pl
from jax.experimental.pallas import tpu as pltpu
from jax.experimental.pallas import tpu_sc as plsc

_NUM_CORES = 2
_NUM_SUBCORES = 16
_NW = _NUM_CORES * _NUM_SUBCORES
_CHUNK = 128


@functools.partial(jax.jit, static_argnums=())
def _gather(labels_r, log_probs):
    nw, n_ch, ch = labels_r.shape
    _, d = log_probs.shape
    mesh = plsc.VectorSubcoreMesh(core_axis_name="c", subcore_axis_name="s")

    @functools.partial(
        pl.kernel,
        mesh=mesh,
        out_type=jax.ShapeDtypeStruct((nw, n_ch, ch, d), jnp.float32),
        scratch_types=[
            pltpu.VMEM((n_ch, ch), jnp.int32),
            pltpu.VMEM((n_ch, ch, d), jnp.float32),
            pltpu.SemaphoreType.DMA((n_ch,)),
            pltpu.SemaphoreType.DMA((n_ch,)),
        ],
    )
    def body(labels_hbm, table_hbm, out_hbm, idx_v, rows_v, gsem, ssem):
        wid = lax.axis_index("s") * _NUM_CORES + lax.axis_index("c")
        pltpu.sync_copy(labels_hbm.at[wid], idx_v)
        gathers = [
            pltpu.async_copy(table_hbm.at[idx_v.at[j]], rows_v.at[j], gsem.at[j])
            for j in range(n_ch)
        ]
        stores = []
        for j in range(n_ch):
            gathers[j].wait()
            stores.append(
                pltpu.async_copy(rows_v.at[j], out_hbm.at[wid].at[j], ssem.at[j])
            )
        for s in stores:
            s.wait()

    return body(labels_r, log_probs)


def kernel(labels, log_probs):
    (b,) = labels.shape
    _, d = log_probs.shape
    b_per_w = b // _NW
    n_ch = b_per_w // _CHUNK
    labels_r = labels.astype(jnp.int32).reshape(_NW, n_ch, _CHUNK)
    out = _gather(labels_r, log_probs)
    return out.reshape(b, d)


# speedup vs baseline: 1.0052x; 1.0052x over previous
import functools

import jax
import jax.numpy as jnp
from jax import lax
from jax.experimental import pallas as pl
from jax.experimental.pallas import tpu as pltpu
from jax.experimental.pallas import tpu_sc as plsc

_NUM_CORES = 2
_NUM_SUBCORES = 16
_NW = _NUM_CORES * _NUM_SUBCORES
_CHUNK = 128


def _gather(labels, log_probs):
    (b,) = labels.shape
    _, d = log_probs.shape
    b_per_w = b // _NW
    n_ch = b_per_w // _CHUNK
    mesh = plsc.VectorSubcoreMesh(core_axis_name="c", subcore_axis_name="s")

    @functools.partial(
        pl.kernel,
        mesh=mesh,
        out_type=jax.ShapeDtypeStruct((b, d), jnp.float32),
        scratch_types=[
            pltpu.VMEM((n_ch * _CHUNK,), jnp.int32),
            pltpu.VMEM((n_ch, _CHUNK, d), jnp.float32),
            pltpu.SemaphoreType.DMA((n_ch,)),
            pltpu.SemaphoreType.DMA((n_ch,)),
        ],
    )
    def body(labels_hbm, table_hbm, out_hbm, idx_v, rows_v, gsem, ssem):
        wid = lax.axis_index("s") * _NUM_CORES + lax.axis_index("c")
        base = wid * b_per_w
        pltpu.sync_copy(labels_hbm.at[pl.ds(base, b_per_w)], idx_v)
        gathers = [
            pltpu.async_copy(
                table_hbm.at[idx_v.at[pl.ds(j * _CHUNK, _CHUNK)]],
                rows_v.at[j],
                gsem.at[j],
            )
            for j in range(n_ch)
        ]
        stores = []
        for j in range(n_ch):
            gathers[j].wait()
            stores.append(
                pltpu.async_copy(
                    rows_v.at[j],
                    out_hbm.at[pl.ds(base + j * _CHUNK, _CHUNK)],
                    ssem.at[j],
                )
            )
        for s in stores:
            s.wait()

    return body(labels, log_probs)


def kernel(labels, log_probs):
    return _gather(labels.astype(jnp.int32), log_probs)


# speedup vs baseline: 1.0065x; 1.0013x over previous
import functools

import jax
import jax.numpy as jnp
from jax import lax
from jax.experimental import pallas as pl
from jax.experimental.pallas import tpu as pltpu
from jax.experimental.pallas import tpu_sc as plsc

_NUM_CORES = 2
_NUM_SUBCORES = 16
_NW = _NUM_CORES * _NUM_SUBCORES
_CHUNK = 128


def _gather(labels, log_probs):
    (b,) = labels.shape
    _, d = log_probs.shape
    b_per_w = b // _NW
    n_ch = b_per_w // _CHUNK
    mesh = plsc.VectorSubcoreMesh(core_axis_name="c", subcore_axis_name="s")

    @functools.partial(
        pl.kernel,
        mesh=mesh,
        out_type=jax.ShapeDtypeStruct((b, d), jnp.float32),
        scratch_types=[
            pltpu.VMEM((n_ch * _CHUNK,), jnp.int32),
            pltpu.VMEM((n_ch, _CHUNK, d), jnp.float32),
            pltpu.SemaphoreType.DMA((n_ch,)),
            pltpu.SemaphoreType.DMA((n_ch,)),
        ],
    )
    def body(labels_hbm, table_hbm, out_hbm, idx_v, rows_v, gsem, ssem):
        wid = lax.axis_index("s") * _NUM_CORES + lax.axis_index("c")
        base = wid * b_per_w
        pltpu.sync_copy(labels_hbm.at[pl.ds(base, b_per_w)], idx_v)

        def fire(j, _):
            pltpu.async_copy(
                table_hbm.at[idx_v.at[pl.ds(j * _CHUNK, _CHUNK)]],
                rows_v.at[j],
                gsem.at[j],
            )
            return 0

        def drain_fire(j, _):
            pltpu.make_async_copy(
                table_hbm.at[idx_v.at[pl.ds(j * _CHUNK, _CHUNK)]],
                rows_v.at[j],
                gsem.at[j],
            ).wait()
            pltpu.async_copy(
                rows_v.at[j],
                out_hbm.at[pl.ds(base + j * _CHUNK, _CHUNK)],
                ssem.at[j],
            )
            return 0

        def drain_store(j, _):
            pltpu.make_async_copy(
                rows_v.at[j],
                out_hbm.at[pl.ds(base + j * _CHUNK, _CHUNK)],
                ssem.at[j],
            ).wait()
            return 0

        lax.fori_loop(0, n_ch, fire, 0)
        lax.fori_loop(0, n_ch, drain_fire, 0)
        lax.fori_loop(0, n_ch, drain_store, 0)

    return body(labels, log_probs)


def kernel(labels, log_probs):
    return _gather(labels.astype(jnp.int32), log_probs)
